# baseline (speedup 1.0000x reference)
import functools

import jax
import jax.numpy as jnp
from jax import lax
from jax.experimental import pallas as pl
from jax.experimental.pallas import tpu as pltpu
from jax.experimental.pallas import tpu_sc as plsc

_LANES = 16


def _make_gather(B, D):
    info = plsc.get_sparse_core_info()
    NC, NS = info.num_cores, info.num_subcores
    NW = NC * NS
    assert B % (NW * _LANES) == 0
    b_per_w = B // NW
    n_chunks = b_per_w // 128
    n_rounds = 2
    rows_buf = b_per_w // n_rounds
    mesh = plsc.VectorSubcoreMesh(core_axis_name="c", subcore_axis_name="s")

    @functools.partial(
        pl.kernel,
        out_type=jax.ShapeDtypeStruct((B // 4, 4 * D), jnp.float32),
        mesh=mesh,
        scratch_types=[
            pltpu.VMEM((b_per_w,), jnp.int32),
            pltpu.VMEM((n_chunks, 128), jnp.int32),
            pltpu.VMEM((rows_buf, 4 * D), jnp.float32),
            pltpu.VMEM((b_per_w // 4, 4 * D), jnp.float32),
            pltpu.SemaphoreType.DMA,
        ],
        compiler_params=pltpu.CompilerParams(needs_layout_passes=False),
    )
    def gather_kernel(idx_hbm, table_hbm, out_hbm, idx_v, rid_v, rows_v,
                      out_v, sem):
        wid = lax.axis_index("s") * NC + lax.axis_index("c")
        base = wid * b_per_w
        pltpu.sync_copy(idx_hbm.at[pl.ds(base, b_per_w)], idx_v)

        for j in range(b_per_w // _LANES):
            chunk = idx_v[pl.ds(j * _LANES, _LANES)] >> 2
            rid_v[j // 8, pl.ds((j % 8) * _LANES, _LANES)] = chunk

        lane_ids = lax.iota(jnp.int32, _LANES)
        blocks_per_round = rows_buf // _LANES
        chunks_per_round = n_chunks // n_rounds

        for r in range(n_rounds):
            copies = [
                pltpu.async_copy(
                    table_hbm.at[rid_v.at[r * chunks_per_round + k]],
                    rows_v.at[pl.ds(k * 128, 128)],
                    sem,
                )
                for k in range(chunks_per_round)
            ]
            for c in copies:
                c.wait()

            def block_body(i, carry):
                loc16 = lane_ids + i * _LANES
                glob16 = loc16 + r * rows_buf
                idx16 = idx_v[pl.ds(i * _LANES + r * rows_buf, _LANES)]
                src_off = (idx16 & 3) << 5
                dst_row = glob16 >> 2
                dst_off = (glob16 & 3) << 5
                for c in range(D):
                    colswz = (lane_ids + c) & (D - 1)
                    vals = plsc.load_gather(rows_v, [loc16, src_off + colswz])
                    plsc.store_scatter(out_v, [dst_row, dst_off + colswz],
                                       vals)
                return carry

            lax.fori_loop(0, blocks_per_round, block_body, 0, unroll=2)

        pltpu.sync_copy(out_v, out_hbm.at[pl.ds(wid * (b_per_w // 4),
                                                b_per_w // 4)])

    return gather_kernel


def kernel(task_ids, table):
    (B,) = task_ids.shape
    V, D = table.shape
    table4 = table.reshape(V // 4, 4 * D)
    out4 = _make_gather(B, D)(task_ids.astype(jnp.int32), table4)
    return out4.reshape(B, D)


# speedup vs baseline: 1.5380x; 1.5380x over previous
import functools

import jax
import jax.numpy as jnp
from jax import lax
from jax.experimental import pallas as pl
from jax.experimental.pallas import tpu as pltpu
from jax.experimental.pallas import tpu_sc as plsc

_LANES = 16


def _make_gather(B, D):
    info = plsc.get_sparse_core_info()
    NC, NS = info.num_cores, info.num_subcores
    NW = NC * NS
    assert B % (NW * _LANES) == 0
    b_per_w = B // NW
    mesh = plsc.VectorSubcoreMesh(core_axis_name="c", subcore_axis_name="s")

    @functools.partial(
        pl.kernel,
        out_type=jax.ShapeDtypeStruct((B, D), jnp.float32),
        mesh=mesh,
        scratch_types=[
            pltpu.VMEM((b_per_w,), jnp.int32),
            pltpu.VMEM((b_per_w, D), jnp.float32),
            pltpu.SemaphoreType.DMA,
        ],
    )
    def gather_kernel(idx_hbm, table_hbm, out_hbm, idx_v, rows_v, sem):
        wid = lax.axis_index("s") * NC + lax.axis_index("c")
        base = wid * b_per_w
        pltpu.sync_copy(idx_hbm.at[pl.ds(base, b_per_w)], idx_v)

        def block_body(i, carry):
            idx16 = idx_v[pl.ds(i * _LANES, _LANES)]
            for j in range(_LANES):
                pltpu.async_copy(
                    table_hbm.at[pl.ds(idx16[j], 1)],
                    rows_v.at[pl.ds(i * _LANES + j, 1)],
                    sem,
                )
            return carry

        lax.fori_loop(0, b_per_w // _LANES, block_body, 0, unroll=2)

        pltpu.make_async_copy(
            table_hbm.at[pl.ds(0, b_per_w)], rows_v, sem
        ).wait()

        pltpu.sync_copy(rows_v, out_hbm.at[pl.ds(base, b_per_w)])

    return gather_kernel


def kernel(task_ids, table):
    (B,) = task_ids.shape
    V, D = table.shape
    return _make_gather(B, D)(task_ids.astype(jnp.int32), table)
